# baseline (speedup 1.0000x reference)
import functools

import jax
import jax.numpy as jnp
from jax import lax
from jax.experimental import pallas as pl
from jax.experimental.pallas import tpu as pltpu
from jax.experimental.pallas import tpu_sc as plsc


def _sigmoid_body(x_ref, o_ref):
    o_ref[...] = jax.nn.sigmoid(x_ref[...])


def _sigmoid_table(logits):
    return pl.pallas_call(
        _sigmoid_body,
        out_shape=jax.ShapeDtypeStruct(logits.shape, logits.dtype),
    )(logits)


@functools.cache
def _make_gather(n_rows, d, b):
    nc, ns = 2, 16
    nw = nc * ns
    b_per_w = b // nw
    chunk = 64
    n_chunks = b_per_w // chunk
    mesh = plsc.VectorSubcoreMesh(core_axis_name="c", subcore_axis_name="s")

    @functools.partial(
        pl.kernel,
        mesh=mesh,
        out_type=jax.ShapeDtypeStruct((b, d), jnp.float32),
        scratch_types=[
            pltpu.VMEM((b_per_w,), jnp.int32),
            pltpu.VMEM((chunk, d), jnp.float32),
            pltpu.SemaphoreType.DMA,
        ],
        compiler_params=pltpu.CompilerParams(use_tc_tiling_on_sc=False),
    )
    def gather_kernel(table_hbm, idx_hbm, out_hbm, idx_v, rows_v, sem):
        wid = lax.axis_index("s") * nc + lax.axis_index("c")
        base = wid * b_per_w
        pltpu.sync_copy(idx_hbm.at[pl.ds(base, b_per_w)], idx_v)

        def body(j, carry):
            off = pl.multiple_of(j * chunk, 8)
            pltpu.async_copy(
                table_hbm.at[idx_v.at[pl.ds(off, chunk)]], rows_v, sem
            ).wait()
            pltpu.sync_copy(rows_v, out_hbm.at[pl.ds(base + off, chunk)])
            return carry

        lax.fori_loop(0, n_chunks, body, 0)

    return gather_kernel


def kernel(indices, logits):
    n_rows, d = logits.shape
    (b,) = indices.shape
    sig_table = _sigmoid_table(logits)
    return _make_gather(n_rows, d, b)(sig_table, indices)


# speedup vs baseline: 1.0155x; 1.0155x over previous
import functools

import jax
import jax.numpy as jnp
from jax import lax
from jax.experimental import pallas as pl
from jax.experimental.pallas import tpu as pltpu
from jax.experimental.pallas import tpu_sc as plsc


def _sigmoid_body(x_ref, o_ref):
    o_ref[...] = jax.nn.sigmoid(x_ref[...])


def _sigmoid_table(logits):
    return pl.pallas_call(
        _sigmoid_body,
        out_shape=jax.ShapeDtypeStruct(logits.shape, logits.dtype),
    )(logits)


@functools.cache
def _make_gather(n_rows, d, b):
    nc, ns = 2, 16
    nw = nc * ns
    b_per_w = b // nw
    chunk = 64
    n_chunks = b_per_w // chunk
    mesh = plsc.VectorSubcoreMesh(core_axis_name="c", subcore_axis_name="s")

    @functools.partial(
        pl.kernel,
        mesh=mesh,
        out_type=jax.ShapeDtypeStruct((b, d), jnp.float32),
        scratch_types=[
            pltpu.VMEM((b_per_w,), jnp.int32),
            pltpu.VMEM((chunk, d), jnp.float32),
            pltpu.VMEM((chunk, d), jnp.float32),
            pltpu.SemaphoreType.DMA,
            pltpu.SemaphoreType.DMA,
            pltpu.SemaphoreType.DMA,
            pltpu.SemaphoreType.DMA,
        ],
        compiler_params=pltpu.CompilerParams(use_tc_tiling_on_sc=False),
    )
    def gather_kernel(table_hbm, idx_hbm, out_hbm, idx_v, rows0, rows1,
                      g0, g1, s0, s1):
        wid = lax.axis_index("s") * nc + lax.axis_index("c")
        base = wid * b_per_w
        pltpu.sync_copy(idx_hbm.at[pl.ds(base, b_per_w)], idx_v)
        rows = (rows0, rows1)
        gsem = (g0, g1)
        ssem = (s0, s1)

        def start_gather(j):
            bsel = j % 2
            return pltpu.async_copy(
                table_hbm.at[idx_v.at[pl.ds(j * chunk, chunk)]],
                rows[bsel], gsem[bsel],
            )

        gathers = [None] * n_chunks
        scatters = [None] * n_chunks
        gathers[0] = start_gather(0)
        for j in range(n_chunks):
            bsel = j % 2
            if j + 1 < n_chunks:
                if j - 1 >= 0:
                    scatters[j - 1].wait()
                gathers[j + 1] = start_gather(j + 1)
            gathers[j].wait()
            scatters[j] = pltpu.async_copy(
                rows[bsel], out_hbm.at[pl.ds(base + j * chunk, chunk)],
                ssem[bsel],
            )
        scatters[n_chunks - 2].wait()
        scatters[n_chunks - 1].wait()

    return gather_kernel


def kernel(indices, logits):
    n_rows, d = logits.shape
    (b,) = indices.shape
    sig_table = _sigmoid_table(logits)
    return _make_gather(n_rows, d, b)(sig_table, indices)


# speedup vs baseline: 1.0231x; 1.0074x over previous
import functools

import jax
import jax.numpy as jnp
from jax import lax
from jax.experimental import pallas as pl
from jax.experimental.pallas import tpu as pltpu
from jax.experimental.pallas import tpu_sc as plsc

_D_PAD = 1024


def _sigmoid_pad_body(x_ref, o_ref):
    d = x_ref.shape[1]
    o_ref[:, :d] = jax.nn.sigmoid(x_ref[...])


def _sigmoid_table_padded(logits):
    n_rows, _ = logits.shape
    return pl.pallas_call(
        _sigmoid_pad_body,
        out_shape=jax.ShapeDtypeStruct((n_rows, _D_PAD), logits.dtype),
    )(logits)


@functools.cache
def _make_gather(n_rows, d, b):
    nc, ns = 2, 16
    nw = nc * ns
    b_per_w = b // nw
    chunk = 32
    n_chunks = b_per_w // chunk
    mesh = plsc.VectorSubcoreMesh(core_axis_name="c", subcore_axis_name="s")

    @functools.partial(
        pl.kernel,
        mesh=mesh,
        out_type=jax.ShapeDtypeStruct((b, d), jnp.float32),
        scratch_types=[
            pltpu.VMEM((b_per_w,), jnp.int32),
            pltpu.VMEM((chunk, _D_PAD), jnp.float32),
            pltpu.VMEM((chunk, _D_PAD), jnp.float32),
            pltpu.SemaphoreType.DMA,
            pltpu.SemaphoreType.DMA,
            pltpu.SemaphoreType.DMA,
            pltpu.SemaphoreType.DMA,
        ],
        compiler_params=pltpu.CompilerParams(use_tc_tiling_on_sc=False),
    )
    def gather_kernel(table_hbm, idx_hbm, out_hbm, idx_v, rows0, rows1,
                      g0, g1, s0, s1):
        wid = lax.axis_index("s") * nc + lax.axis_index("c")
        base = wid * b_per_w
        pltpu.sync_copy(idx_hbm.at[pl.ds(base, b_per_w)], idx_v)
        rows = (rows0, rows1)
        gsem = (g0, g1)
        ssem = (s0, s1)

        def start_gather(j):
            bsel = j % 2
            return pltpu.async_copy(
                table_hbm.at[idx_v.at[pl.ds(j * chunk, chunk)]],
                rows[bsel], gsem[bsel],
            )

        gathers = [None] * n_chunks
        scatters = [None] * n_chunks
        gathers[0] = start_gather(0)
        for j in range(n_chunks):
            bsel = j % 2
            if j + 1 < n_chunks:
                if j - 1 >= 0:
                    scatters[j - 1].wait()
                gathers[j + 1] = start_gather(j + 1)
            gathers[j].wait()
            scatters[j] = pltpu.async_copy(
                rows[bsel].at[:, pl.ds(0, d)],
                out_hbm.at[pl.ds(base + j * chunk, chunk)],
                ssem[bsel],
            )
        scatters[n_chunks - 2].wait()
        scatters[n_chunks - 1].wait()

    return gather_kernel


def kernel(indices, logits):
    n_rows, d = logits.shape
    (b,) = indices.shape
    sig_table = _sigmoid_table_padded(logits)
    return _make_gather(n_rows, d, b)(sig_table, indices)


# speedup vs baseline: 1.4817x; 1.4483x over previous
import functools

import jax
import jax.numpy as jnp
from jax import lax
from jax.experimental import pallas as pl
from jax.experimental.pallas import tpu as pltpu
from jax.experimental.pallas import tpu_sc as plsc

_D_PAD = 1024


def _sigmoid_pad_body(x_ref, o_ref):
    d = x_ref.shape[1]
    o_ref[:, :d] = jax.nn.sigmoid(x_ref[...])


def _sigmoid_table_padded(logits):
    n_rows, _ = logits.shape
    return pl.pallas_call(
        _sigmoid_pad_body,
        out_shape=jax.ShapeDtypeStruct((n_rows, _D_PAD), logits.dtype),
    )(logits)


@functools.cache
def _make_gather(n_rows, d, b):
    nc, ns = 2, 16
    nw = nc * ns
    b_per_w = b // nw
    chunk = 32
    n_chunks = b_per_w // chunk
    d_main = (d // 128) * 128
    d_tail = d - d_main
    n_tail_full = d_tail // 16
    tail_rem = d_tail - n_tail_full * 16
    mesh = plsc.VectorSubcoreMesh(core_axis_name="c", subcore_axis_name="s")

    @functools.partial(
        pl.kernel,
        mesh=mesh,
        out_type=jax.ShapeDtypeStruct((b, d), jnp.float32),
        scratch_types=[
            pltpu.VMEM((b_per_w,), jnp.int32),
            pltpu.VMEM((chunk, d), jnp.float32),
            pltpu.VMEM((chunk, d), jnp.float32),
            pltpu.VMEM((chunk, 128), jnp.float32),
            pltpu.VMEM((chunk, 128), jnp.float32),
            pltpu.SemaphoreType.DMA,
            pltpu.SemaphoreType.DMA,
            pltpu.SemaphoreType.DMA,
            pltpu.SemaphoreType.DMA,
        ],
        compiler_params=pltpu.CompilerParams(
            use_tc_tiling_on_sc=True, needs_layout_passes=False),
    )
    def gather_kernel(table_hbm, idx_hbm, out_hbm, idx_v, rows0, rows1,
                      tail0, tail1, g0, g1, s0, s1):
        wid = lax.axis_index("s") * nc + lax.axis_index("c")
        base = wid * b_per_w
        pltpu.sync_copy(idx_hbm.at[pl.ds(base, b_per_w)], idx_v)
        rows = (rows0, rows1)
        tails = (tail0, tail1)
        gsem = (g0, g1)
        ssem = (s0, s1)

        def start_gathers(j):
            bsel = j % 2
            idx_slice = idx_v.at[pl.ds(j * chunk, chunk)]
            main = pltpu.async_copy(
                table_hbm.at[idx_slice, pl.ds(0, d_main)],
                rows[bsel].at[:, pl.ds(0, d_main)], gsem[bsel],
            )
            tail = pltpu.async_copy(
                table_hbm.at[idx_slice, pl.ds(d_main, 128)],
                tails[bsel], gsem[bsel],
            )
            return main, tail

        def fix_tail(bsel):
            rows_b, tail_b = rows[bsel], tails[bsel]
            col_idx = d_main + n_tail_full * 16 + lax.iota(jnp.int32, 16)
            msk = lax.iota(jnp.int32, 16) < tail_rem

            def body(r, carry):
                for k in range(n_tail_full):
                    rows_b[r, pl.ds(d_main + 16 * k, 16)] = (
                        tail_b[r, pl.ds(16 * k, 16)])
                v = tail_b[r, pl.ds(n_tail_full * 16, 16)]
                row_idx = jnp.full((16,), r, dtype=jnp.int32)
                plsc.store_scatter(rows_b, [row_idx, col_idx], v, mask=msk)
                return carry

            lax.fori_loop(0, chunk, body, 0)

        gathers = [None] * n_chunks
        scatters = [None] * n_chunks
        gathers[0] = start_gathers(0)
        for j in range(n_chunks):
            bsel = j % 2
            if j + 1 < n_chunks:
                if j - 1 >= 0:
                    scatters[j - 1].wait()
                gathers[j + 1] = start_gathers(j + 1)
            gathers[j][0].wait()
            gathers[j][1].wait()
            fix_tail(bsel)
            scatters[j] = pltpu.async_copy(
                rows[bsel], out_hbm.at[pl.ds(base + j * chunk, chunk)],
                ssem[bsel],
            )
        scatters[n_chunks - 2].wait()
        scatters[n_chunks - 1].wait()

    return gather_kernel


def kernel(indices, logits):
    n_rows, d = logits.shape
    (b,) = indices.shape
    sig_table = _sigmoid_table_padded(logits)
    return _make_gather(n_rows, d, b)(sig_table, indices)
